# baseline (speedup 1.0000x reference)
import functools

import jax
import jax.numpy as jnp
from jax import lax
from jax.experimental import pallas as pl
from jax.experimental.pallas import tpu as pltpu
from jax.experimental.pallas import tpu_sc as plsc

_NC = 2
_NS = 16
_NW = _NC * _NS
_CHUNK = 128
_RING = 4


def _rsqrt(x):
    y = plsc.bitcast(jnp.int32(0x5F3759DF) - (plsc.bitcast(x, jnp.int32) >> 1),
                     jnp.float32)
    xh = x * 0.5
    for _ in range(3):
        y = y * (1.5 - xh * y * y)
    return y


def _sc_body(cpw, x_hbm, f0_hbm, f1_hbm, f2_hbm, zero_hbm, out0_hbm, out1_hbm,
             acc, iv, gv, rv, ov, isem, gsem, ssem):
    c_ax = lax.axis_index("c")
    s_ax = lax.axis_index("s")

    @pl.when(s_ax == 0)
    def _():
        pltpu.sync_copy(zero_hbm, acc)

    plsc.subcore_barrier()

    w = c_ax * _NS + s_ax
    f_hbm = (f0_hbm, f1_hbm, f2_hbm)

    def chunk_base(c):
        return pl.multiple_of((w * cpw + c) * _CHUNK, _CHUNK)

    def idx_issue(c, b):
        base = chunk_base(c)
        for v in range(3):
            pltpu.async_copy(f_hbm[v].at[pl.ds(base, _CHUNK)], iv[b][v], isem[b])

    def idx_wait(c, b):
        base = chunk_base(c)
        for v in range(3):
            pltpu.make_async_copy(f_hbm[v].at[pl.ds(base, _CHUNK)], iv[b][v],
                                  isem[b]).wait()

    def build_issue(b):
        for j in range(_CHUNK // 16):
            sl = pl.ds(j * 16, 16)
            for v in range(3):
                i3 = iv[b][v][sl] * 3
                gv[b][3 * v + 0][sl] = i3
                gv[b][3 * v + 1][sl] = i3 + 1
                gv[b][3 * v + 2][sl] = i3 + 2
        for t in range(9):
            pltpu.async_copy(x_hbm.at[gv[b][t]], rv[b][t], gsem[b])

    def gwait(b):
        for t in range(9):
            pltpu.make_async_copy(x_hbm.at[gv[b][t]], rv[b][t], gsem[b]).wait()

    def compute(b):
        for j in range(_CHUNK // 16):
            sl = pl.ds(j * 16, 16)
            v1 = [rv[b][t][sl] for t in range(3)]
            v2 = [rv[b][3 + t][sl] for t in range(3)]
            v3 = [rv[b][6 + t][sl] for t in range(3)]
            d1 = [v2[t] - v3[t] for t in range(3)]
            d2 = [v3[t] - v1[t] for t in range(3)]
            d3 = [v1[t] - v2[t] for t in range(3)]
            q1 = d1[0] * d1[0] + d1[1] * d1[1] + d1[2] * d1[2]
            q2 = d2[0] * d2[0] + d2[1] * d2[1] + d2[2] * d2[2]
            q3 = d3[0] * d3[0] + d3[1] * d3[1] + d3[2] * d3[2]
            l1 = q1 * _rsqrt(q1)
            l2 = q2 * _rsqrt(q2)
            l3 = q3 * _rsqrt(q3)
            sp = (l1 + l2 + l3) * 0.5
            ins = sp * (sp - l1) * (sp - l2) * (sp - l3)
            ins = jnp.maximum(ins, 0.0)
            area2 = 2.0 * (ins * _rsqrt(ins))
            recip = 0.25 / (area2 + 1e-10)
            recip = jnp.where(area2 == 0.0, 0.0, recip)
            w0 = (q2 + q3 - q1) * recip
            w1 = (q1 + q3 - q2) * recip
            w2 = (q1 + q2 - q3) * recip
            for t in range(3):
                ov[b][t][sl] = w1 * d2[t] - w2 * d3[t]
                ov[b][3 + t][sl] = w2 * d3[t] - w0 * d1[t]
                ov[b][6 + t][sl] = w0 * d1[t] - w1 * d2[t]

    def scatter_issue(b):
        for t in range(9):
            pltpu.async_copy(ov[b][t], acc.at[gv[b][t]], ssem[b], add=True)

    def swait(b):
        for t in range(9):
            pltpu.make_async_copy(ov[b][t], acc.at[gv[b][t]], ssem[b]).wait()

    for b in range(3):
        idx_issue(b, b)
    for b in range(2):
        idx_wait(b, b)
        build_issue(b)

    def super_body(ks, carry):
        for ph in range(_RING):
            c = ks * _RING + ph

            @pl.when(c + 3 < cpw)
            def _():
                idx_issue(c + 3, (ph + 3) % _RING)

            @pl.when(c + 2 < cpw)
            def _():
                @pl.when(c >= 2)
                def _():
                    swait((ph + 2) % _RING)

                idx_wait(c + 2, (ph + 2) % _RING)
                build_issue((ph + 2) % _RING)

            gwait(ph)
            compute(ph)
            scatter_issue(ph)
        return carry

    lax.fori_loop(0, cpw // _RING, super_body, 0)
    for b in range(_RING):
        swait(b)
    plsc.subcore_barrier()

    @pl.when(s_ax == 0)
    def _():
        @pl.when(c_ax == 0)
        def _():
            pltpu.sync_copy(acc, out0_hbm)

        @pl.when(c_ax == 1)
        def _():
            pltpu.sync_copy(acc, out1_hbm)


def _combine_body(a_ref, b_ref, o_ref):
    o_ref[...] = a_ref[...] + b_ref[...]


@jax.jit
def kernel(V, F):
    B, N, _ = V.shape
    Fn = F.shape[1]
    BN = B * N
    T = B * Fn
    cpw = _RING * (-(-T // (_NW * _CHUNK * _RING)))
    TP = _NW * cpw * _CHUNK

    x = V.reshape(BN * 3)
    offs = (jnp.arange(B, dtype=F.dtype) * jnp.asarray(N, F.dtype))[:, None, None]
    bf = (F + offs).reshape(T, 3)
    pad = TP - T
    f0 = jnp.concatenate([bf[:, 0], jnp.zeros((pad,), bf.dtype)])
    f1 = jnp.concatenate([bf[:, 1], jnp.zeros((pad,), bf.dtype)])
    f2 = jnp.concatenate([bf[:, 2], jnp.zeros((pad,), bf.dtype)])
    zero = jnp.zeros((BN * 3,), jnp.float32)

    mesh = plsc.VectorSubcoreMesh(core_axis_name="c", subcore_axis_name="s",
                                  num_cores=_NC, num_subcores=_NS)
    sc_call = pl.kernel(
        functools.partial(_sc_body, cpw),
        out_type=(jax.ShapeDtypeStruct((BN * 3,), jnp.float32),
                  jax.ShapeDtypeStruct((BN * 3,), jnp.float32)),
        mesh=mesh,
        scratch_types=[
            pltpu.VMEM_SHARED((BN * 3,), jnp.float32),
            [[pltpu.VMEM((_CHUNK,), jnp.int32) for _ in range(3)]
             for _ in range(_RING)],
            [[pltpu.VMEM((_CHUNK,), jnp.int32) for _ in range(9)]
             for _ in range(_RING)],
            [[pltpu.VMEM((_CHUNK,), jnp.float32) for _ in range(9)]
             for _ in range(_RING)],
            [[pltpu.VMEM((_CHUNK,), jnp.float32) for _ in range(9)]
             for _ in range(_RING)],
            [pltpu.SemaphoreType.DMA for _ in range(_RING)],
            [pltpu.SemaphoreType.DMA for _ in range(_RING)],
            [pltpu.SemaphoreType.DMA for _ in range(_RING)],
        ],
        compiler_params=pltpu.CompilerParams(needs_layout_passes=False),
    )
    p0, p1 = sc_call(x, f0, f1, f2, zero)

    L = BN * 3
    Lp = -(-L // 512) * 512
    q0 = jnp.pad(p0, (0, Lp - L)).reshape(-1, 512)
    q1 = jnp.pad(p1, (0, Lp - L)).reshape(-1, 512)
    out = pl.pallas_call(
        _combine_body,
        out_shape=jax.ShapeDtypeStruct(q0.shape, jnp.float32),
    )(q0, q1)
    return out.reshape(-1)[:L].reshape(B, N, 3)


# speedup vs baseline: 5.3471x; 5.3471x over previous
import functools

import jax
import jax.numpy as jnp
from jax import lax
from jax.experimental import pallas as pl
from jax.experimental.pallas import tpu as pltpu
from jax.experimental.pallas import tpu_sc as plsc

_NC = 2
_NS = 16
_NW = _NC * _NS
_CHUNK = 128
_RING = 4


def _rsqrt(x):
    y = plsc.bitcast(jnp.int32(0x5F3759DF) - (plsc.bitcast(x, jnp.int32) >> 1),
                     jnp.float32)
    xh = x * 0.5
    for _ in range(3):
        y = y * (1.5 - xh * y * y)
    return y


def _sc_body(cpw, xp0, xp1, xp2, f0_hbm, f1_hbm, f2_hbm, zero_hbm,
             out0_hbm, out1_hbm, acc, iv, rv, ov, isem, gsem, ssem):
    c_ax = lax.axis_index("c")
    s_ax = lax.axis_index("s")
    bnp = zero_hbm.shape[0]

    @pl.when(s_ax == 0)
    def _():
        for t in range(3):
            pltpu.sync_copy(zero_hbm, acc[t])

    plsc.subcore_barrier()

    w = c_ax * _NS + s_ax
    f_hbm = (f0_hbm, f1_hbm, f2_hbm)
    xp = (xp0, xp1, xp2)

    def chunk_base(c):
        return pl.multiple_of((w * cpw + c) * _CHUNK, _CHUNK)

    def idx_issue(c, b):
        base = chunk_base(c)
        for v in range(3):
            pltpu.async_copy(f_hbm[v].at[pl.ds(base, _CHUNK)], iv[b][v], isem[b])

    def idx_wait(c, b):
        base = chunk_base(c)
        for v in range(3):
            pltpu.make_async_copy(f_hbm[v].at[pl.ds(base, _CHUNK)], iv[b][v],
                                  isem[b]).wait()

    def gather_issue(b):
        for v in range(3):
            for t in range(3):
                pltpu.async_copy(xp[t].at[iv[b][v]], rv[b][3 * v + t], gsem[b])

    def gwait(b):
        for v in range(3):
            for t in range(3):
                pltpu.make_async_copy(xp[t].at[iv[b][v]], rv[b][3 * v + t],
                                      gsem[b]).wait()

    def compute(b):
        for j in range(_CHUNK // 16):
            sl = pl.ds(j * 16, 16)
            v1 = [rv[b][t][sl] for t in range(3)]
            v2 = [rv[b][3 + t][sl] for t in range(3)]
            v3 = [rv[b][6 + t][sl] for t in range(3)]
            d1 = [v2[t] - v3[t] for t in range(3)]
            d2 = [v3[t] - v1[t] for t in range(3)]
            d3 = [v1[t] - v2[t] for t in range(3)]
            q1 = d1[0] * d1[0] + d1[1] * d1[1] + d1[2] * d1[2]
            q2 = d2[0] * d2[0] + d2[1] * d2[1] + d2[2] * d2[2]
            q3 = d3[0] * d3[0] + d3[1] * d3[1] + d3[2] * d3[2]
            l1 = q1 * _rsqrt(q1)
            l2 = q2 * _rsqrt(q2)
            l3 = q3 * _rsqrt(q3)
            sp = (l1 + l2 + l3) * 0.5
            ins = sp * (sp - l1) * (sp - l2) * (sp - l3)
            ins = jnp.maximum(ins, 0.0)
            area2 = 2.0 * (ins * _rsqrt(ins))
            recip = 0.25 / (area2 + 1e-10)
            recip = jnp.where(area2 == 0.0, 0.0, recip)
            w0 = (q2 + q3 - q1) * recip
            w1 = (q1 + q3 - q2) * recip
            w2 = (q1 + q2 - q3) * recip
            for t in range(3):
                ov[b][t][sl] = w1 * d2[t] - w2 * d3[t]
                ov[b][3 + t][sl] = w2 * d3[t] - w0 * d1[t]
                ov[b][6 + t][sl] = w0 * d1[t] - w1 * d2[t]

    def scatter_issue(b):
        for v in range(3):
            for t in range(3):
                pltpu.async_copy(ov[b][3 * v + t], acc[t].at[iv[b][v]],
                                 ssem[b], add=True)

    def swait(b):
        for v in range(3):
            for t in range(3):
                pltpu.make_async_copy(ov[b][3 * v + t], acc[t].at[iv[b][v]],
                                      ssem[b]).wait()

    for b in range(3):
        idx_issue(b, b)
    for b in range(2):
        idx_wait(b, b)
        gather_issue(b)

    def super_body(ks, carry):
        for ph in range(_RING):
            c = ks * _RING + ph

            @pl.when(c + 3 < cpw)
            def _():
                idx_issue(c + 3, (ph + 3) % _RING)

            @pl.when(c + 2 < cpw)
            def _():
                @pl.when(c >= 2)
                def _():
                    swait((ph + 2) % _RING)

                idx_wait(c + 2, (ph + 2) % _RING)
                gather_issue((ph + 2) % _RING)

            gwait(ph)
            compute(ph)
            scatter_issue(ph)
        return carry

    lax.fori_loop(0, cpw // _RING, super_body, 0)
    for b in range(_RING):
        swait(b)
    plsc.subcore_barrier()

    @pl.when(s_ax == 0)
    def _():
        @pl.when(c_ax == 0)
        def _():
            for t in range(3):
                pltpu.sync_copy(acc[t], out0_hbm.at[pl.ds(t * bnp, bnp)])

        @pl.when(c_ax == 1)
        def _():
            for t in range(3):
                pltpu.sync_copy(acc[t], out1_hbm.at[pl.ds(t * bnp, bnp)])


def _combine_body(a_ref, b_ref, o_ref):
    o_ref[...] = a_ref[...] + b_ref[...]


@jax.jit
def kernel(V, F):
    B, N, _ = V.shape
    Fn = F.shape[1]
    BN = B * N
    T = B * Fn
    cpw = _RING * (-(-T // (_NW * _CHUNK * _RING)))
    TP = _NW * cpw * _CHUNK

    planes = jnp.moveaxis(V, 2, 0)
    xp0 = planes[0].reshape(BN)
    xp1 = planes[1].reshape(BN)
    xp2 = planes[2].reshape(BN)

    offs = (jnp.arange(B, dtype=F.dtype) * jnp.asarray(N, F.dtype))[:, None]
    fp = jnp.moveaxis(F, 2, 0)
    pad = TP - T
    pad_idx = jnp.arange(pad, dtype=F.dtype) % jnp.asarray(BN, F.dtype)
    f0 = jnp.concatenate([(fp[0] + offs).reshape(T), pad_idx])
    f1 = jnp.concatenate([(fp[1] + offs).reshape(T), pad_idx])
    f2 = jnp.concatenate([(fp[2] + offs).reshape(T), pad_idx])
    BNP = -(-BN // 512) * 512
    zero = jnp.zeros((BNP,), jnp.float32)

    mesh = plsc.VectorSubcoreMesh(core_axis_name="c", subcore_axis_name="s",
                                  num_cores=_NC, num_subcores=_NS)
    sc_call = pl.kernel(
        functools.partial(_sc_body, cpw),
        out_type=(jax.ShapeDtypeStruct((3 * BNP,), jnp.float32),
                  jax.ShapeDtypeStruct((3 * BNP,), jnp.float32)),
        mesh=mesh,
        scratch_types=[
            [pltpu.VMEM_SHARED((BNP,), jnp.float32) for _ in range(3)],
            [[pltpu.VMEM((_CHUNK,), jnp.int32) for _ in range(3)]
             for _ in range(_RING)],
            [[pltpu.VMEM((_CHUNK,), jnp.float32) for _ in range(9)]
             for _ in range(_RING)],
            [[pltpu.VMEM((_CHUNK,), jnp.float32) for _ in range(9)]
             for _ in range(_RING)],
            [pltpu.SemaphoreType.DMA for _ in range(_RING)],
            [pltpu.SemaphoreType.DMA for _ in range(_RING)],
            [pltpu.SemaphoreType.DMA for _ in range(_RING)],
        ],
        compiler_params=pltpu.CompilerParams(needs_layout_passes=False),
    )
    p0, p1 = sc_call(xp0, xp1, xp2, f0, f1, f2, zero)

    q0 = p0.reshape(-1, 512)
    q1 = p1.reshape(-1, 512)
    out = pl.pallas_call(
        _combine_body,
        out_shape=jax.ShapeDtypeStruct(q0.shape, jnp.float32),
    )(q0, q1)
    res = out.reshape(3, BNP)[:, :BN].reshape(3, B, N)
    return jnp.moveaxis(res, 0, 2)


# speedup vs baseline: 6.4448x; 1.2053x over previous
import functools

import jax
import jax.numpy as jnp
from jax import lax
from jax.experimental import pallas as pl
from jax.experimental.pallas import tpu as pltpu
from jax.experimental.pallas import tpu_sc as plsc

_NC = 2
_NS = 16
_NW = _NC * _NS
_CHUNK = 128
_RING = 6


def _rsqrt(x):
    y = plsc.bitcast(jnp.int32(0x5F3759DF) - (plsc.bitcast(x, jnp.int32) >> 1),
                     jnp.float32)
    xh = x * 0.5
    for _ in range(3):
        y = y * (1.5 - xh * y * y)
    return y


def _sc_body(cpw, xp0, xp1, xp2, f0_hbm, f1_hbm, f2_hbm, zero_hbm,
             out0_hbm, out1_hbm, acc, iv, rv, ov, isem, gsem, ssem):
    c_ax = lax.axis_index("c")
    s_ax = lax.axis_index("s")
    bnp = zero_hbm.shape[0]

    @pl.when(s_ax == 0)
    def _():
        for t in range(3):
            pltpu.sync_copy(zero_hbm, acc[t])

    plsc.subcore_barrier()

    w = c_ax * _NS + s_ax
    f_hbm = (f0_hbm, f1_hbm, f2_hbm)
    xp = (xp0, xp1, xp2)

    def chunk_base(c):
        return pl.multiple_of((w * cpw + c) * _CHUNK, _CHUNK)

    def idx_issue(c, b):
        base = chunk_base(c)
        for v in range(3):
            pltpu.async_copy(f_hbm[v].at[pl.ds(base, _CHUNK)], iv[b][v], isem[b])

    def idx_wait(c, b):
        base = chunk_base(c)
        for v in range(3):
            pltpu.make_async_copy(f_hbm[v].at[pl.ds(base, _CHUNK)], iv[b][v],
                                  isem[b]).wait()

    def gather_issue(b):
        for v in range(3):
            for t in range(3):
                pltpu.async_copy(xp[t].at[iv[b][v]], rv[b][3 * v + t], gsem[b])

    def gwait(b):
        for v in range(3):
            for t in range(3):
                pltpu.make_async_copy(xp[t].at[iv[b][v]], rv[b][3 * v + t],
                                      gsem[b]).wait()

    def compute(b):
        for j in range(_CHUNK // 16):
            sl = pl.ds(j * 16, 16)
            v1 = [rv[b][t][sl] for t in range(3)]
            v2 = [rv[b][3 + t][sl] for t in range(3)]
            v3 = [rv[b][6 + t][sl] for t in range(3)]
            d1 = [v2[t] - v3[t] for t in range(3)]
            d2 = [v3[t] - v1[t] for t in range(3)]
            d3 = [v1[t] - v2[t] for t in range(3)]
            q1 = d1[0] * d1[0] + d1[1] * d1[1] + d1[2] * d1[2]
            q2 = d2[0] * d2[0] + d2[1] * d2[1] + d2[2] * d2[2]
            q3 = d3[0] * d3[0] + d3[1] * d3[1] + d3[2] * d3[2]
            l1 = q1 * _rsqrt(q1)
            l2 = q2 * _rsqrt(q2)
            l3 = q3 * _rsqrt(q3)
            sp = (l1 + l2 + l3) * 0.5
            ins = sp * (sp - l1) * (sp - l2) * (sp - l3)
            ins = jnp.maximum(ins, 0.0)
            area2 = 2.0 * (ins * _rsqrt(ins))
            recip = 0.25 / (area2 + 1e-10)
            recip = jnp.where(area2 == 0.0, 0.0, recip)
            w0 = (q2 + q3 - q1) * recip
            w1 = (q1 + q3 - q2) * recip
            w2 = (q1 + q2 - q3) * recip
            for t in range(3):
                ov[b][t][sl] = w1 * d2[t] - w2 * d3[t]
                ov[b][3 + t][sl] = w2 * d3[t] - w0 * d1[t]
                ov[b][6 + t][sl] = w0 * d1[t] - w1 * d2[t]

    def scatter_issue(b):
        for v in range(3):
            for t in range(3):
                pltpu.async_copy(ov[b][3 * v + t], acc[t].at[iv[b][v]],
                                 ssem[b], add=True)

    def swait(b):
        for v in range(3):
            for t in range(3):
                pltpu.make_async_copy(ov[b][3 * v + t], acc[t].at[iv[b][v]],
                                      ssem[b]).wait()

    for b in range(3):
        idx_issue(b, b)
    for b in range(2):
        idx_wait(b, b)
        gather_issue(b)

    def super_body(ks, carry):
        for ph in range(_RING):
            c = ks * _RING + ph

            @pl.when(c + 3 < cpw)
            def _():
                @pl.when(c >= 3)
                def _():
                    swait((ph + 3) % _RING)

                idx_issue(c + 3, (ph + 3) % _RING)

            @pl.when(c + 2 < cpw)
            def _():
                idx_wait(c + 2, (ph + 2) % _RING)
                gather_issue((ph + 2) % _RING)

            gwait(ph)
            compute(ph)
            scatter_issue(ph)
        return carry

    nsup = cpw // _RING
    lax.fori_loop(0, nsup, super_body, 0)
    for c in range(nsup * _RING, cpw):
        ph = c % _RING
        if c + 3 < cpw:
            if c >= 3:
                swait((ph + 3) % _RING)
            idx_issue(c + 3, (ph + 3) % _RING)
        if c + 2 < cpw:
            idx_wait(c + 2, (ph + 2) % _RING)
            gather_issue((ph + 2) % _RING)
        gwait(ph)
        compute(ph)
        scatter_issue(ph)
    for b in range(_RING):
        swait(b)
    plsc.subcore_barrier()

    @pl.when(s_ax == 0)
    def _():
        @pl.when(c_ax == 0)
        def _():
            for t in range(3):
                pltpu.sync_copy(acc[t], out0_hbm.at[pl.ds(t * bnp, bnp)])

        @pl.when(c_ax == 1)
        def _():
            for t in range(3):
                pltpu.sync_copy(acc[t], out1_hbm.at[pl.ds(t * bnp, bnp)])


def _combine_body(a_ref, b_ref, o_ref):
    o_ref[...] = a_ref[...] + b_ref[...]


@jax.jit
def kernel(V, F):
    B, N, _ = V.shape
    Fn = F.shape[1]
    BN = B * N
    T = B * Fn
    cpw = -(-T // (_NW * _CHUNK))
    TP = _NW * cpw * _CHUNK

    planes = jnp.moveaxis(V, 2, 0)
    xp0 = planes[0].reshape(BN)
    xp1 = planes[1].reshape(BN)
    xp2 = planes[2].reshape(BN)

    offs = (jnp.arange(B, dtype=F.dtype) * jnp.asarray(N, F.dtype))[:, None]
    fp = jnp.moveaxis(F, 2, 0)
    pad = TP - T
    pad_idx = jnp.arange(pad, dtype=F.dtype) % jnp.asarray(BN, F.dtype)
    f0 = jnp.concatenate([(fp[0] + offs).reshape(T), pad_idx])
    f1 = jnp.concatenate([(fp[1] + offs).reshape(T), pad_idx])
    f2 = jnp.concatenate([(fp[2] + offs).reshape(T), pad_idx])
    BNP = -(-BN // 512) * 512
    zero = jnp.zeros((BNP,), jnp.float32)

    mesh = plsc.VectorSubcoreMesh(core_axis_name="c", subcore_axis_name="s",
                                  num_cores=_NC, num_subcores=_NS)
    sc_call = pl.kernel(
        functools.partial(_sc_body, cpw),
        out_type=(jax.ShapeDtypeStruct((3 * BNP,), jnp.float32),
                  jax.ShapeDtypeStruct((3 * BNP,), jnp.float32)),
        mesh=mesh,
        scratch_types=[
            [pltpu.VMEM_SHARED((BNP,), jnp.float32) for _ in range(3)],
            [[pltpu.VMEM((_CHUNK,), jnp.int32) for _ in range(3)]
             for _ in range(_RING)],
            [[pltpu.VMEM((_CHUNK,), jnp.float32) for _ in range(9)]
             for _ in range(_RING)],
            [[pltpu.VMEM((_CHUNK,), jnp.float32) for _ in range(9)]
             for _ in range(_RING)],
            [pltpu.SemaphoreType.DMA for _ in range(_RING)],
            [pltpu.SemaphoreType.DMA for _ in range(_RING)],
            [pltpu.SemaphoreType.DMA for _ in range(_RING)],
        ],
        compiler_params=pltpu.CompilerParams(needs_layout_passes=False),
    )
    p0, p1 = sc_call(xp0, xp1, xp2, f0, f1, f2, zero)

    q0 = p0.reshape(-1, 512)
    q1 = p1.reshape(-1, 512)
    out = pl.pallas_call(
        _combine_body,
        out_shape=jax.ShapeDtypeStruct(q0.shape, jnp.float32),
    )(q0, q1)
    res = out.reshape(3, BNP)[:, :BN].reshape(3, B, N)
    return jnp.moveaxis(res, 0, 2)


# speedup vs baseline: 6.5999x; 1.0241x over previous
import functools

import jax
import jax.numpy as jnp
from jax import lax
from jax.experimental import pallas as pl
from jax.experimental.pallas import tpu as pltpu
from jax.experimental.pallas import tpu_sc as plsc

_NC = 2
_NS = 16
_NW = _NC * _NS
_CHUNK = 128
_RING = 6


def _rsqrt(x):
    y = plsc.bitcast(jnp.int32(0x5F3759DF) - (plsc.bitcast(x, jnp.int32) >> 1),
                     jnp.float32)
    xh = x * 0.5
    for _ in range(2):
        y = y * (1.5 - xh * y * y)
    return y


def _sc_body(cpw, xp0, xp1, xp2, f0_hbm, f1_hbm, f2_hbm, zero_hbm,
             out0_hbm, out1_hbm, acc, iv, rv, ov, isem, gsem, ssem):
    c_ax = lax.axis_index("c")
    s_ax = lax.axis_index("s")
    bnp = zero_hbm.shape[0]

    for t in range(3):
        @pl.when(s_ax == t)
        def _(t=t):
            pltpu.sync_copy(zero_hbm, acc[t])

    plsc.subcore_barrier()

    w = c_ax * _NS + s_ax
    f_hbm = (f0_hbm, f1_hbm, f2_hbm)
    xp = (xp0, xp1, xp2)

    def chunk_base(c):
        return pl.multiple_of((w * cpw + c) * _CHUNK, _CHUNK)

    def idx_issue(c, b):
        base = chunk_base(c)
        for v in range(3):
            pltpu.async_copy(f_hbm[v].at[pl.ds(base, _CHUNK)], iv[b][v], isem[b])

    def idx_wait(c, b):
        base = chunk_base(c)
        for v in range(3):
            pltpu.make_async_copy(f_hbm[v].at[pl.ds(base, _CHUNK)], iv[b][v],
                                  isem[b]).wait()

    def gather_issue(b):
        for v in range(3):
            for t in range(3):
                pltpu.async_copy(xp[t].at[iv[b][v]], rv[b][3 * v + t], gsem[b])

    def gwait(b):
        for v in range(3):
            for t in range(3):
                pltpu.make_async_copy(xp[t].at[iv[b][v]], rv[b][3 * v + t],
                                      gsem[b]).wait()

    def compute(b):
        for j in range(_CHUNK // 16):
            sl = pl.ds(j * 16, 16)
            v1 = [rv[b][t][sl] for t in range(3)]
            v2 = [rv[b][3 + t][sl] for t in range(3)]
            v3 = [rv[b][6 + t][sl] for t in range(3)]
            d1 = [v2[t] - v3[t] for t in range(3)]
            d2 = [v3[t] - v1[t] for t in range(3)]
            d3 = [v1[t] - v2[t] for t in range(3)]
            q1 = d1[0] * d1[0] + d1[1] * d1[1] + d1[2] * d1[2]
            q2 = d2[0] * d2[0] + d2[1] * d2[1] + d2[2] * d2[2]
            q3 = d3[0] * d3[0] + d3[1] * d3[1] + d3[2] * d3[2]
            l1 = q1 * _rsqrt(q1)
            l2 = q2 * _rsqrt(q2)
            l3 = q3 * _rsqrt(q3)
            sp = (l1 + l2 + l3) * 0.5
            ins = sp * (sp - l1) * (sp - l2) * (sp - l3)
            ins = jnp.maximum(ins, 0.0)
            area2 = 2.0 * (ins * _rsqrt(ins))
            recip = 0.25 / (area2 + 1e-10)
            recip = jnp.where(area2 == 0.0, 0.0, recip)
            w0 = (q2 + q3 - q1) * recip
            w1 = (q1 + q3 - q2) * recip
            w2 = (q1 + q2 - q3) * recip
            for t in range(3):
                ov[b][t][sl] = w1 * d2[t] - w2 * d3[t]
                ov[b][3 + t][sl] = w2 * d3[t] - w0 * d1[t]
                ov[b][6 + t][sl] = w0 * d1[t] - w1 * d2[t]

    def scatter_issue(b):
        for v in range(3):
            for t in range(3):
                pltpu.async_copy(ov[b][3 * v + t], acc[t].at[iv[b][v]],
                                 ssem[b], add=True)

    def swait(b):
        for v in range(3):
            for t in range(3):
                pltpu.make_async_copy(ov[b][3 * v + t], acc[t].at[iv[b][v]],
                                      ssem[b]).wait()

    for b in range(3):
        idx_issue(b, b)
    for b in range(2):
        idx_wait(b, b)
        gather_issue(b)

    def super_body(ks, carry):
        for ph in range(_RING):
            c = ks * _RING + ph

            @pl.when(c + 3 < cpw)
            def _():
                @pl.when(c >= 3)
                def _():
                    swait((ph + 3) % _RING)

                idx_issue(c + 3, (ph + 3) % _RING)

            @pl.when(c + 2 < cpw)
            def _():
                idx_wait(c + 2, (ph + 2) % _RING)
                gather_issue((ph + 2) % _RING)

            gwait(ph)
            compute(ph)
            scatter_issue(ph)
        return carry

    nsup = cpw // _RING
    lax.fori_loop(0, nsup, super_body, 0)
    for c in range(nsup * _RING, cpw):
        ph = c % _RING
        if c + 3 < cpw:
            if c >= 3:
                swait((ph + 3) % _RING)
            idx_issue(c + 3, (ph + 3) % _RING)
        if c + 2 < cpw:
            idx_wait(c + 2, (ph + 2) % _RING)
            gather_issue((ph + 2) % _RING)
        gwait(ph)
        compute(ph)
        scatter_issue(ph)
    for b in range(_RING):
        swait(b)
    plsc.subcore_barrier()

    for t in range(3):
        @pl.when(s_ax == t)
        def _(t=t):
            @pl.when(c_ax == 0)
            def _():
                pltpu.sync_copy(acc[t], out0_hbm.at[pl.ds(t * bnp, bnp)])

            @pl.when(c_ax == 1)
            def _():
                pltpu.sync_copy(acc[t], out1_hbm.at[pl.ds(t * bnp, bnp)])


def _combine_body(a_ref, b_ref, o_ref):
    o_ref[...] = a_ref[...] + b_ref[...]


@jax.jit
def kernel(V, F):
    B, N, _ = V.shape
    Fn = F.shape[1]
    BN = B * N
    T = B * Fn
    cpw = -(-T // (_NW * _CHUNK))
    TP = _NW * cpw * _CHUNK

    planes = jnp.moveaxis(V, 2, 0)
    xp0 = planes[0].reshape(BN)
    xp1 = planes[1].reshape(BN)
    xp2 = planes[2].reshape(BN)

    offs = (jnp.arange(B, dtype=F.dtype) * jnp.asarray(N, F.dtype))[:, None]
    fp = jnp.moveaxis(F, 2, 0)
    pad = TP - T
    pad_idx = jnp.arange(pad, dtype=F.dtype) % jnp.asarray(BN, F.dtype)
    f0 = jnp.concatenate([(fp[0] + offs).reshape(T), pad_idx])
    f1 = jnp.concatenate([(fp[1] + offs).reshape(T), pad_idx])
    f2 = jnp.concatenate([(fp[2] + offs).reshape(T), pad_idx])
    BNP = -(-BN // 512) * 512
    zero = jnp.zeros((BNP,), jnp.float32)

    mesh = plsc.VectorSubcoreMesh(core_axis_name="c", subcore_axis_name="s",
                                  num_cores=_NC, num_subcores=_NS)
    sc_call = pl.kernel(
        functools.partial(_sc_body, cpw),
        out_type=(jax.ShapeDtypeStruct((3 * BNP,), jnp.float32),
                  jax.ShapeDtypeStruct((3 * BNP,), jnp.float32)),
        mesh=mesh,
        scratch_types=[
            [pltpu.VMEM_SHARED((BNP,), jnp.float32) for _ in range(3)],
            [[pltpu.VMEM((_CHUNK,), jnp.int32) for _ in range(3)]
             for _ in range(_RING)],
            [[pltpu.VMEM((_CHUNK,), jnp.float32) for _ in range(9)]
             for _ in range(_RING)],
            [[pltpu.VMEM((_CHUNK,), jnp.float32) for _ in range(9)]
             for _ in range(_RING)],
            [pltpu.SemaphoreType.DMA for _ in range(_RING)],
            [pltpu.SemaphoreType.DMA for _ in range(_RING)],
            [pltpu.SemaphoreType.DMA for _ in range(_RING)],
        ],
        compiler_params=pltpu.CompilerParams(needs_layout_passes=False),
    )
    p0, p1 = sc_call(xp0, xp1, xp2, f0, f1, f2, zero)

    q0 = p0.reshape(-1, 512)
    q1 = p1.reshape(-1, 512)
    out = pl.pallas_call(
        _combine_body,
        out_shape=jax.ShapeDtypeStruct(q0.shape, jnp.float32),
    )(q0, q1)
    res = out.reshape(3, BNP)[:, :BN].reshape(3, B, N)
    return jnp.moveaxis(res, 0, 2)


# speedup vs baseline: 6.7185x; 1.0180x over previous
import functools

import jax
import jax.numpy as jnp
from jax import lax
from jax.experimental import pallas as pl
from jax.experimental.pallas import tpu as pltpu
from jax.experimental.pallas import tpu_sc as plsc

_NC = 2
_NS = 16
_NW = _NC * _NS
_CHUNK = 128
_RING = 6


def _rsqrt(x):
    y = plsc.bitcast(jnp.int32(0x5F3759DF) - (plsc.bitcast(x, jnp.int32) >> 1),
                     jnp.float32)
    xh = x * 0.5
    for _ in range(2):
        y = y * (1.5 - xh * y * y)
    return y


def _sc_body(cpw, xp0, xp1, xp2, f0_hbm, f1_hbm, f2_hbm, zero_hbm,
             out0_hbm, out1_hbm, acc, iv, rv, ov, isem, gsem, ssem):
    c_ax = lax.axis_index("c")
    s_ax = lax.axis_index("s")
    bnp = zero_hbm.shape[0]

    for t in range(3):
        @pl.when(s_ax == t)
        def _(t=t):
            pltpu.sync_copy(zero_hbm, acc[t])

    plsc.subcore_barrier()

    w = c_ax * _NS + s_ax
    f_hbm = (f0_hbm, f1_hbm, f2_hbm)
    xp = (xp0, xp1, xp2)

    def chunk_base(c):
        return pl.multiple_of((w * cpw + c) * _CHUNK, _CHUNK)

    def idx_issue(c, b):
        base = chunk_base(c)
        for v in range(3):
            pltpu.async_copy(f_hbm[v].at[pl.ds(base, _CHUNK)], iv[b][v], isem[b])

    def idx_wait(c, b):
        base = chunk_base(c)
        for v in range(3):
            pltpu.make_async_copy(f_hbm[v].at[pl.ds(base, _CHUNK)], iv[b][v],
                                  isem[b]).wait()

    def gather_issue(b):
        for v in range(3):
            for t in range(3):
                pltpu.async_copy(xp[t].at[iv[b][v]], rv[b][3 * v + t], gsem[b])

    def gwait(b):
        for v in range(3):
            for t in range(3):
                pltpu.make_async_copy(xp[t].at[iv[b][v]], rv[b][3 * v + t],
                                      gsem[b]).wait()

    def compute(b):
        for j in range(_CHUNK // 16):
            sl = pl.ds(j * 16, 16)
            v1 = [rv[b][t][sl] for t in range(3)]
            v2 = [rv[b][3 + t][sl] for t in range(3)]
            v3 = [rv[b][6 + t][sl] for t in range(3)]
            d1 = [v2[t] - v3[t] for t in range(3)]
            d2 = [v3[t] - v1[t] for t in range(3)]
            d3 = [v1[t] - v2[t] for t in range(3)]
            q1 = d1[0] * d1[0] + d1[1] * d1[1] + d1[2] * d1[2]
            q2 = d2[0] * d2[0] + d2[1] * d2[1] + d2[2] * d2[2]
            q3 = d3[0] * d3[0] + d3[1] * d3[1] + d3[2] * d3[2]
            l1 = q1 * _rsqrt(q1)
            l2 = q2 * _rsqrt(q2)
            l3 = q3 * _rsqrt(q3)
            sp = (l1 + l2 + l3) * 0.5
            ins = sp * (sp - l1) * (sp - l2) * (sp - l3)
            ins = jnp.maximum(ins, 0.0)
            area2 = 2.0 * (ins * _rsqrt(ins))
            recip = 0.25 / (area2 + 1e-10)
            recip = jnp.where(area2 == 0.0, 0.0, recip)
            w0 = (q2 + q3 - q1) * recip
            w1 = (q1 + q3 - q2) * recip
            w2 = (q1 + q2 - q3) * recip
            for t in range(3):
                ov[b][t][sl] = w1 * d2[t] - w2 * d3[t]
                ov[b][3 + t][sl] = w2 * d3[t] - w0 * d1[t]
                ov[b][6 + t][sl] = w0 * d1[t] - w1 * d2[t]

    def scatter_issue(b):
        for v in range(3):
            for t in range(3):
                pltpu.async_copy(ov[b][3 * v + t], acc[t].at[iv[b][v]],
                                 ssem[b], add=True)

    def swait(b):
        for v in range(3):
            for t in range(3):
                pltpu.make_async_copy(ov[b][3 * v + t], acc[t].at[iv[b][v]],
                                      ssem[b]).wait()

    for b in range(3):
        idx_issue(b, b)
    for b in range(2):
        idx_wait(b, b)
        gather_issue(b)

    def super_body(ks, carry):
        for ph in range(_RING):
            c = ks * _RING + ph

            @pl.when(c + 3 < cpw)
            def _():
                @pl.when(c >= 3)
                def _():
                    swait((ph + 3) % _RING)

                idx_issue(c + 3, (ph + 3) % _RING)

            @pl.when(c + 2 < cpw)
            def _():
                idx_wait(c + 2, (ph + 2) % _RING)
                gather_issue((ph + 2) % _RING)

            gwait(ph)
            compute(ph)
            scatter_issue(ph)
        return carry

    nsup = cpw // _RING
    lax.fori_loop(0, nsup, super_body, 0)
    for c in range(nsup * _RING, cpw):
        ph = c % _RING
        if c + 3 < cpw:
            if c >= 3:
                swait((ph + 3) % _RING)
            idx_issue(c + 3, (ph + 3) % _RING)
        if c + 2 < cpw:
            idx_wait(c + 2, (ph + 2) % _RING)
            gather_issue((ph + 2) % _RING)
        gwait(ph)
        compute(ph)
        scatter_issue(ph)
    for b in range(_RING):
        swait(b)
    plsc.subcore_barrier()

    for t in range(3):
        @pl.when(s_ax == t)
        def _(t=t):
            @pl.when(c_ax == 0)
            def _():
                pltpu.sync_copy(acc[t], out0_hbm.at[pl.ds(t * bnp, bnp)])

            @pl.when(c_ax == 1)
            def _():
                pltpu.sync_copy(acc[t], out1_hbm.at[pl.ds(t * bnp, bnp)])


def _combine_body(a_ref, b_ref, o_ref):
    o_ref[...] = a_ref[...] + b_ref[...]


@jax.jit
def kernel(V, F):
    B, N, _ = V.shape
    Fn = F.shape[1]
    BN = B * N
    T = B * Fn
    cpw = -(-T // (_NW * _CHUNK))
    TP = _NW * cpw * _CHUNK

    planes = jnp.moveaxis(V, 2, 0)
    xp0 = planes[0].reshape(BN)
    xp1 = planes[1].reshape(BN)
    xp2 = planes[2].reshape(BN)

    offs = (jnp.arange(B, dtype=F.dtype) * jnp.asarray(N, F.dtype))[:, None]
    fp = jnp.moveaxis(F, 2, 0)
    pad = TP - T
    pad_idx = jnp.arange(pad, dtype=F.dtype) % jnp.asarray(BN, F.dtype)
    f0 = jnp.concatenate([(fp[0] + offs).reshape(T), pad_idx])
    f1 = jnp.concatenate([(fp[1] + offs).reshape(T), pad_idx])
    f2 = jnp.concatenate([(fp[2] + offs).reshape(T), pad_idx])
    BNP = -(-BN // 512) * 512
    zero = jnp.zeros((BNP,), jnp.float32)

    mesh = plsc.VectorSubcoreMesh(core_axis_name="c", subcore_axis_name="s",
                                  num_cores=_NC, num_subcores=_NS)
    sc_call = pl.kernel(
        functools.partial(_sc_body, cpw),
        out_type=(jax.ShapeDtypeStruct((3 * BNP,), jnp.float32),
                  jax.ShapeDtypeStruct((3 * BNP,), jnp.float32)),
        mesh=mesh,
        scratch_types=[
            [pltpu.VMEM_SHARED((BNP,), jnp.float32) for _ in range(3)],
            [[pltpu.VMEM((_CHUNK,), jnp.int32) for _ in range(3)]
             for _ in range(_RING)],
            [[pltpu.VMEM((_CHUNK,), jnp.float32) for _ in range(9)]
             for _ in range(_RING)],
            [[pltpu.VMEM((_CHUNK,), jnp.float32) for _ in range(9)]
             for _ in range(_RING)],
            [pltpu.SemaphoreType.DMA for _ in range(_RING)],
            [pltpu.SemaphoreType.DMA for _ in range(_RING)],
            [pltpu.SemaphoreType.DMA for _ in range(_RING)],
        ],
        compiler_params=pltpu.CompilerParams(needs_layout_passes=False),
    )
    p0, p1 = sc_call(xp0, xp1, xp2, f0, f1, f2, zero)

    out = pl.pallas_call(
        _combine_body,
        out_shape=jax.ShapeDtypeStruct(p0.shape, jnp.float32),
    )(p0, p1)
    res = out.reshape(3, BNP)[:, :BN].reshape(3, B, N)
    return jnp.moveaxis(res, 0, 2)
